# baseline (speedup 1.0000x reference)
import functools

import jax
import jax.numpy as jnp
from jax import lax
from jax.experimental import pallas as pl
from jax.experimental.pallas import tpu as pltpu
from jax.experimental.pallas import tpu_sc as plsc

_D = 32
_E = 200000
_NW = 32
_CHUNK = 128
_NPH = 2
_CHP = 25
_EPH = _NW * _CHP * _CHUNK
_EPAD = _NPH * _EPH
_BT = 2048
_BR = _BT // 4
_GBP = _EPH // _BT


def _sc_gather(node_emb, head_idx, tail_idx):
    mesh = plsc.VectorSubcoreMesh(core_axis_name="c", subcore_axis_name="s")

    @functools.partial(
        pl.kernel,
        mesh=mesh,
        out_type=jax.ShapeDtypeStruct((_NW, _CHP, _CHUNK * _D // 128, 128),
                                      jnp.float32),
        scratch_types=[
            pltpu.VMEM((_CHP, _CHUNK), jnp.int32),
            pltpu.VMEM((_CHP, _CHUNK), jnp.int32),
            pltpu.VMEM((_CHUNK, _D), jnp.float32),
            pltpu.VMEM((_CHUNK, _D), jnp.float32),
            pltpu.VMEM((_CHUNK * _D // 128, 128), jnp.float32),
            pltpu.SemaphoreType.DMA,
            pltpu.SemaphoreType.DMA,
        ],
        compiler_params=pltpu.CompilerParams(use_tc_tiling_on_sc=False),
    )
    def gather_kernel(node_hbm, hidx_hbm, tidx_hbm, dout_hbm,
                      hidx_v, tidx_v, hbuf, tbuf, dbuf, sem_h, sem_t):
        wid = lax.axis_index("s") * 2 + lax.axis_index("c")
        pltpu.sync_copy(hidx_hbm.at[wid], hidx_v)
        pltpu.sync_copy(tidx_hbm.at[wid], tidx_v)

        def body(c, carry):
            cp_h = pltpu.async_copy(node_hbm.at[hidx_v.at[c]], hbuf, sem_h)
            cp_t = pltpu.async_copy(node_hbm.at[tidx_v.at[c]], tbuf, sem_t)
            cp_h.wait()
            cp_t.wait()

            def sub_vec(k, carry2):
                v = (hbuf[lax.shift_right_logical(k, 1), pl.ds((k & 1) * 16, 16)]
                     - tbuf[lax.shift_right_logical(k, 1), pl.ds((k & 1) * 16, 16)])
                dbuf[lax.shift_right_logical(k, 3), pl.ds((k & 7) * 16, 16)] = v
                return carry2

            lax.fori_loop(0, _CHUNK * _D // 16, sub_vec, 0)
            pltpu.sync_copy(dbuf, dout_hbm.at[wid, c])
            return carry

        lax.fori_loop(0, _CHP, body, 0)

    return gather_kernel(node_emb, head_idx, tail_idx)


def _tc_partial(diff2d, ridx3, p_stacked, r_emb_w, e_off):
    def body(d_ref, r_ref, p_ref, e_ref, o_ref):
        i = pl.program_id(0)
        blk = d_ref[...]

        trow = lax.broadcasted_iota(jnp.int32, (_D, _D * _D), 0)
        tcol = lax.broadcasted_iota(jnp.int32, (_D, _D * _D), 1)
        tmat = ((tcol & (_D - 1)) == trow).astype(jnp.float32)
        col = lax.broadcasted_iota(jnp.int32, (_BR, _D * _D), 1)
        gcol = lax.shift_right_logical(col, 5)
        rcol = lax.broadcasted_iota(jnp.int32, (_BR, 64), 1)
        krow = lax.broadcasted_iota(jnp.int32, (_BR, 1), 0)

        part = jnp.zeros((), jnp.float32)
        for j in range(4):
            dj = blk[:, _D * j:_D * (j + 1)]
            rj = r_ref[0, j, :]
            g = lax.shift_right_logical(rj, 1)

            diff_t = jnp.dot(dj, tmat, preferred_element_type=jnp.float32)
            sel = (gcol == g[:, None])
            x = jnp.where(sel, diff_t, 0.0)
            out = jnp.dot(x, p_ref[...], preferred_element_type=jnp.float32)

            onehot_r = (rcol == rj[:, None]).astype(jnp.float32)
            r_e = jnp.dot(onehot_r, e_ref[...],
                          preferred_element_type=jnp.float32)

            s = out + r_e
            e_glob = 4 * (i * _BR + krow) + j + e_off
            s = jnp.where(e_glob < _E, s, 0.0)
            part = part + jnp.sum(s * s)

        @pl.when(i == 0)
        def _init():
            o_ref[...] = jnp.zeros((1, 1), jnp.float32)

        o_ref[...] = o_ref[...] + part

    return pl.pallas_call(
        body,
        grid=(_GBP,),
        in_specs=[
            pl.BlockSpec((_BR, 128), lambda i: (i, 0)),
            pl.BlockSpec((1, 8, _BR), lambda i: (i, 0, 0)),
            pl.BlockSpec((_D * _D, _D), lambda i: (0, 0)),
            pl.BlockSpec((64, _D), lambda i: (0, 0)),
        ],
        out_specs=pl.BlockSpec((1, 1), lambda i: (0, 0)),
        out_shape=jax.ShapeDtypeStruct((1, 1), jnp.float32),
    )(diff2d, ridx3, p_stacked, r_emb_w)


def kernel(node_emb, r_emb_w, r_proj_w, edge_index_t, edge_attr):
    pad = _EPAD - _E
    head_idx = jnp.concatenate(
        [edge_index_t[:, 0], jnp.zeros((pad,), jnp.int32)]).astype(jnp.int32)
    tail_idx = jnp.concatenate(
        [edge_index_t[:, 1], jnp.zeros((pad,), jnp.int32)]).astype(jnp.int32)
    head_idx = head_idx.reshape(_NPH, _NW, _CHP, _CHUNK)
    tail_idx = tail_idx.reshape(_NPH, _NW, _CHP, _CHUNK)

    ridx = jnp.concatenate(
        [edge_attr[:, 0], jnp.zeros((pad,), jnp.int32)]).astype(jnp.int32)

    p_stacked = r_proj_w.reshape(_D * _D, _D)

    total = None
    for p in range(_NPH):
        diff_rows = _sc_gather(node_emb, head_idx[p], tail_idx[p])
        rp = ridx[p * _EPH:(p + 1) * _EPH]
        ridx3 = rp.reshape(_GBP, _BR, 4).transpose(0, 2, 1)
        ridx3 = jnp.pad(ridx3, ((0, 0), (0, 4), (0, 0)))
        s = _tc_partial(diff_rows.reshape(_EPH // 4, 128),
                        ridx3, p_stacked, r_emb_w, p * _EPH)
        total = s if total is None else total + s

    return total[0, 0] * (1.0 / (_E * _D))
